# Initial kernel scaffold; baseline (speedup 1.0000x reference)
import jax
import jax.numpy as jnp
from jax.experimental import pallas as pl


def kernel(x, edge_index, W, b, zeroth_w, fst_w, snd_w, trd_w):
    raise NotImplementedError("write your pallas kernel here")


# speedup vs baseline: 5.7519x; 5.7519x over previous
import jax
import jax.numpy as jnp
from jax import lax
from jax.experimental import pallas as pl
from jax.experimental.pallas import tpu as pltpu
from jax.experimental.pallas import tpu_sc as plsc

N = 10000
E = 160000
E2 = 2 * E
D = 128
NC, NS, L = 2, 16, 16
NW = NC * NS
HALF = N // 2
SLICE = E2 // NW
CH = 128
CAP = 10240
NCHS = CAP // CH
TBL = N * N
DUMP = HALF
ST = 320
RJ = 32

_MESH = dict(
    mesh=plsc.VectorSubcoreMesh(
        core_axis_name="c", subcore_axis_name="s", num_cores=NC, num_subcores=NS),
    compiler_params=pltpu.CompilerParams(needs_layout_passes=False),
)


def _f32(shape):
    return jax.ShapeDtypeStruct(shape, jnp.float32)


def _i32(shape):
    return jax.ShapeDtypeStruct(shape, jnp.int32)


def _extract_i32(vec, idx, iot):
    return lax.reduce_max(jnp.where(iot == idx, vec, 0), axes=(0,))


def _k1a_body(src_hbm, dst_hbm, table_hbm, srcb, dstb, idxb, valb, sem):
    c = lax.axis_index("c")
    s = lax.axis_index("s")
    e0 = (c * NS + s) * SLICE
    swap = c == 1
    iot = lax.iota(jnp.int32, L)
    NB = 8

    pltpu.sync_copy(src_hbm.at[pl.ds(s * SLICE, SLICE)], srcb)
    pltpu.sync_copy(dst_hbm.at[pl.ds(s * SLICE, SLICE)], dstb)

    nchunk = (SLICE + CH - 1) // CH

    def fire(j, slot):
        start = jnp.minimum(j * CH, SLICE - CH)
        for k in range(CH // L):
            off = start + k * L
            sv = srcb[pl.ds(off, L)]
            dv = dstb[pl.ds(off, L)]
            usv = jnp.where(swap, dv, sv)
            udv = jnp.where(swap, sv, dv)
            idxb[slot, pl.ds(k * L, L)] = usv * N + udv
            valb[slot, pl.ds(k * L, L)] = e0 + off + iot
        pltpu.async_copy(valb.at[slot], table_hbm.at[idxb.at[slot]], sem)

    def body(j, carry):
        slot = lax.rem(j, NB)

        @pl.when(j >= NB)
        def _():
            pltpu.make_async_copy(valb.at[slot], table_hbm.at[idxb.at[slot]], sem).wait()

        fire(j, slot)
        return carry

    lax.fori_loop(0, nchunk, body, 0)
    for k in range(NB):
        pltpu.make_async_copy(valb.at[k], table_hbm.at[idxb.at[k]], sem).wait()


_k1a = pl.kernel(
    _k1a_body,
    out_type=_i32((TBL,)),
    scratch_types=[
        pltpu.VMEM((SLICE,), jnp.int32),
        pltpu.VMEM((SLICE,), jnp.int32),
        pltpu.VMEM((8, CH), jnp.int32),
        pltpu.VMEM((8, CH), jnp.int32),
        pltpu.SemaphoreType.DMA,
    ],
    **_MESH,
)


def _k1b_body(src_hbm, dst_hbm, table_hbm, lists_hbm, counts_hbm, hists_hbm,
              srcb, dstb, keyb, winb, listA, listB, hist, cntv, sem):
    c = lax.axis_index("c")
    s = lax.axis_index("s")
    t = c * NS + s
    e0 = t * SLICE
    swap = c == 1
    iot = lax.iota(jnp.int32, L)
    NB = 4
    zero16 = jnp.zeros((L,), jnp.float32)
    one16 = jnp.ones((L,), jnp.float32)

    pltpu.sync_copy(src_hbm.at[pl.ds(s * SLICE, SLICE)], srcb)
    pltpu.sync_copy(dst_hbm.at[pl.ds(s * SLICE, SLICE)], dstb)

    def zh(i, carry):
        hist[pl.ds(i * L, L)] = zero16
        return carry

    lax.fori_loop(0, N // L, zh, 0)

    nchunk = (SLICE + CH - 1) // CH

    def usud(off):
        sv = srcb[pl.ds(off, L)]
        dv = dstb[pl.ds(off, L)]
        usv = jnp.where(swap, dv, sv)
        udv = jnp.where(swap, sv, dv)
        return usv, udv

    def fire(j, slot):
        start = j * CH
        for k in range(CH // L):
            off = start + k * L
            usv, udv = usud(off)
            valid = (off + iot) < SLICE
            keyb[slot, pl.ds(k * L, L)] = jnp.where(valid, usv * N + udv, 0)
        pltpu.async_copy(table_hbm.at[keyb.at[slot]], winb.at[slot], sem)

    for j in range(NB):
        fire(j, j)

    def body(j, carry):
        cntA, cntB = carry
        slot = lax.rem(j, NB)
        pltpu.make_async_copy(table_hbm.at[keyb.at[slot]], winb.at[slot], sem).wait()
        start = j * CH
        for k in range(CH // L):
            off = start + k * L
            usv, udv = usud(off)
            valid = (off + iot) < SLICE
            ev = e0 + off + iot
            winv = winb[slot, pl.ds(k * L, L)]
            leader = valid & (winv == ev) & (usv != udv)
            usv_c = jnp.where(leader, usv, 0)
            plsc.addupdate_scatter(hist, [usv_c], one16, mask=leader)
            in_b = usv >= HALF
            packed = ((usv - jnp.where(in_b, HALF, 0)) << 16) | udv
            mA = leader & jnp.logical_not(in_b)
            mB = leader & in_b
            plsc.store_compressed(listA.at[pl.ds(cntA, L)], packed, mask=mA)
            plsc.store_compressed(listB.at[pl.ds(cntB, L)], packed, mask=mB)
            pcA = plsc.all_reduce_population_count(mA)
            pcB = plsc.all_reduce_population_count(mB)
            if pcA.ndim:
                pcA = lax.reduce_max(pcA, axes=(0,))
                pcB = lax.reduce_max(pcB, axes=(0,))
            cntA = cntA + pcA
            cntB = cntB + pcB

        @pl.when(j + NB < nchunk)
        def _():
            fire(j + NB, slot)

        return cntA, cntB

    cntA, cntB = lax.fori_loop(0, nchunk, body, (jnp.int32(0), jnp.int32(0)))

    cntv[...] = jnp.where(iot == 0, cntA, jnp.where(iot == 1, cntB, 0))
    pltpu.sync_copy(cntv, counts_hbm.at[t])
    pltpu.sync_copy(listA, lists_hbm.at[t, 0])
    pltpu.sync_copy(listB, lists_hbm.at[t, 1])
    pltpu.sync_copy(hist, hists_hbm.at[t])


_k1b = pl.kernel(
    _k1b_body,
    out_type=(_i32((NW, 2, CAP)), _i32((NW, L)), _f32((NW, N))),
    scratch_types=[
        pltpu.VMEM((SLICE,), jnp.int32),
        pltpu.VMEM((SLICE,), jnp.int32),
        pltpu.VMEM((4, CH), jnp.int32),
        pltpu.VMEM((4, CH), jnp.int32),
        pltpu.VMEM((CAP,), jnp.int32),
        pltpu.VMEM((CAP,), jnp.int32),
        pltpu.VMEM((N,), jnp.float32),
        pltpu.VMEM((L,), jnp.int32),
        pltpu.SemaphoreType.DMA,
    ],
    **_MESH,
)


def _k2a_body(zw, fw, sw, tw, x_ref, w_ref, zy_ref, fy_ref, syr_ref):
    z0 = jnp.maximum(zw[0, 0], 0.0)
    f0 = jnp.maximum(fw[0, 0], 0.0)
    s0 = jnp.maximum(sw[0, 0], 0.0)
    t0 = jnp.maximum(tw[0, 0], 0.0)
    tot = z0 + f0 + s0 + t0 + 1e-6
    z, f, sc = z0 / tot, f0 / tot, s0 / tot
    xw = jnp.dot(x_ref[...], w_ref[...], preferred_element_type=jnp.float32)
    y = xw[:, :D]
    yr = xw[:, D:]
    zy_ref[...] = z * y
    fy_ref[...] = f * y
    syr_ref[...] = sc * yr


def _k2a(x, wcat, zw, fw, sw, tw):
    bm = 1000
    return pl.pallas_call(
        _k2a_body,
        grid=(N // bm,),
        in_specs=[
            pl.BlockSpec(memory_space=pltpu.SMEM),
            pl.BlockSpec(memory_space=pltpu.SMEM),
            pl.BlockSpec(memory_space=pltpu.SMEM),
            pl.BlockSpec(memory_space=pltpu.SMEM),
            pl.BlockSpec((bm, D), lambda i: (i, 0)),
            pl.BlockSpec((D, 2 * D), lambda i: (0, 0)),
        ],
        out_specs=[
            pl.BlockSpec((bm, D), lambda i: (i, 0)),
            pl.BlockSpec((bm, D), lambda i: (i, 0)),
            pl.BlockSpec((bm, D), lambda i: (i, 0)),
        ],
        out_shape=[_f32((N, D)), _f32((N, D)), _f32((N, D))],
    )(zw, fw, sw, tw, x, wcat)


def _k2b_body(h_ref, deg_ref, dinv_ref):
    deg = jnp.sum(h_ref[...], axis=0)
    deg_ref[...] = deg
    dinv_ref[...] = lax.rsqrt(deg + 1.0)


def _k2b(hists):
    return pl.pallas_call(
        _k2b_body,
        out_shape=[_f32((N,)), _f32((N,))],
    )(hists)


def _decode_segment(listb, sidx, gidx, cnt, iot):
    nch = (cnt + CH - 1) // CH

    def dec(j, carry):
        for k in range(CH // L):
            off = j * CH + k * L
            pk = listb[pl.ds(off, L)]
            valid = (off + iot) < cnt
            sidx[j, pl.ds(k * L, L)] = jnp.where(valid, pk >> 16, DUMP)
            gidx[j, pl.ds(k * L, L)] = jnp.where(valid, pk & 0xFFFF, 0)
        return carry

    lax.fori_loop(0, nch, dec, 0)
    return nch


def _spmm_segment(tabs, accs, rows_list, sidx, gidx, nch, sem_g, sem_s):
    nt = len(tabs)

    def fire_g(j, slot):
        for i in range(nt):
            pltpu.async_copy(tabs[i].at[gidx.at[j]], rows_list[i].at[slot], sem_g)

    def wait_g(j, slot):
        for i in range(nt):
            pltpu.make_async_copy(tabs[i].at[gidx.at[j]], rows_list[i].at[slot], sem_g).wait()

    def fire_s(j, slot):
        for i in range(nt):
            pltpu.async_copy(rows_list[i].at[slot], accs[i].at[sidx.at[j]], sem_s, add=True)

    def wait_s(j, slot):
        for i in range(nt):
            pltpu.make_async_copy(rows_list[i].at[slot], accs[i].at[sidx.at[j]], sem_s).wait()

    @pl.when(nch > 0)
    def _():
        fire_g(0, 0)

    def body(j, carry):
        slot = lax.rem(j, 2)
        wait_g(j, slot)

        @pl.when(j >= 1)
        def _():
            wait_s(j - 1, 1 - slot)

        @pl.when(j + 1 < nch)
        def _():
            fire_g(j + 1, 1 - slot)

        fire_s(j, slot)
        return carry

    lax.fori_loop(0, nch, body, 0)

    @pl.when(nch > 0)
    def _():
        wait_s(nch - 1, lax.rem(nch - 1, 2))


def _spmm_phase(lists_hbm, counts_hbm, tabs, accs, listb, sidx, gidx, rows_list,
                cntv, sem_g, sem_s, c, s, iot):
    for k in range(2):
        tseg = 2 * s + k
        pltpu.sync_copy(lists_hbm.at[tseg, c], listb)
        pltpu.sync_copy(counts_hbm.at[tseg], cntv)
        cnt = _extract_i32(cntv[...], c, iot)
        nch = _decode_segment(listb, sidx, gidx, cnt, iot)
        _spmm_segment(tabs, accs, rows_list, sidx, gidx, nch, sem_g, sem_s)


def _k3_body(lists_hbm, counts_hbm, fy_hbm, syr_hbm, zer_hbm, u1_hbm, u2_hbm,
             listb, sidx, gidx, rows, cntv, sem_g, sem_s, acc):
    c = lax.axis_index("c")
    s = lax.axis_index("s")
    iot = lax.iota(jnp.int32, L)
    st = jnp.minimum(s * ST, HALF - ST)
    gbase = c * HALF + st

    for tab_hbm, out_hbm in ((fy_hbm, u1_hbm), (syr_hbm, u2_hbm)):
        pltpu.sync_copy(zer_hbm, acc.at[pl.ds(st, ST)])
        plsc.subcore_barrier()
        _spmm_phase(lists_hbm, counts_hbm, [tab_hbm], [acc],
                    listb, sidx, gidx, [rows], cntv, sem_g, sem_s, c, s, iot)
        plsc.subcore_barrier()
        pltpu.sync_copy(acc.at[pl.ds(st, ST)], out_hbm.at[pl.ds(gbase, ST)])
        plsc.subcore_barrier()


_k3 = pl.kernel(
    _k3_body,
    out_type=(_f32((N, D)), _f32((N, D))),
    scratch_types=[
        pltpu.VMEM((CAP,), jnp.int32),
        pltpu.VMEM((NCHS, CH), jnp.int32),
        pltpu.VMEM((NCHS, CH), jnp.int32),
        pltpu.VMEM((2, CH, D), jnp.float32),
        pltpu.VMEM((L,), jnp.int32),
        pltpu.SemaphoreType.DMA,
        pltpu.SemaphoreType.DMA,
        pltpu.VMEM_SHARED((HALF + 8, D), jnp.float32),
    ],
    **_MESH,
)


def _k4_body(lists_hbm, counts_hbm, u2_hbm, zy_hbm, u1_hbm, syr_hbm, deg_hbm,
             dinv_hbm, zer_hbm, cp_hbm, listb, sidx, gidx, rows, u3c, zyc, u1c,
             syrc, cpb, cntv, deg_v, dinv_v, sem_g, sem_s, acc):
    c = lax.axis_index("c")
    s = lax.axis_index("s")
    iot = lax.iota(jnp.int32, L)
    st = jnp.minimum(s * ST, HALF - ST)
    gbase = c * HALF + st

    pltpu.sync_copy(zer_hbm, acc.at[pl.ds(st, ST)])
    plsc.subcore_barrier()
    _spmm_phase(lists_hbm, counts_hbm, [u2_hbm], [acc],
                listb, sidx, gidx, [rows], cntv, sem_g, sem_s, c, s, iot)
    plsc.subcore_barrier()

    pltpu.sync_copy(deg_hbm.at[pl.ds(gbase, ST)], deg_v)
    pltpu.sync_copy(dinv_hbm.at[pl.ds(gbase, ST)], dinv_v)
    zidx = jnp.zeros((L,), jnp.int32)

    def epi(jj, carry):
        pltpu.sync_copy(acc.at[pl.ds(st + jj * RJ, RJ)], u3c)
        pltpu.sync_copy(zy_hbm.at[pl.ds(gbase + jj * RJ, RJ)], zyc)
        pltpu.sync_copy(u1_hbm.at[pl.ds(gbase + jj * RJ, RJ)], u1c)
        pltpu.sync_copy(syr_hbm.at[pl.ds(gbase + jj * RJ, RJ)], syrc)

        def row(r, carry2):
            ri = jj * RJ + r
            dg = plsc.load_gather(deg_v, [zidx + ri])
            dv = plsc.load_gather(dinv_v, [zidx + ri])
            for k in range(D // L):
                sl = pl.ds(k * L, L)
                v = zyc[r, sl] + u1c[r, sl] + u3c[r, sl] - dg * syrc[r, sl]
                cpb[r, sl] = dv * v
            return carry2

        lax.fori_loop(0, RJ, row, 0)
        pltpu.sync_copy(cpb, cp_hbm.at[pl.ds(gbase + jj * RJ, RJ)])
        return carry

    lax.fori_loop(0, ST // RJ, epi, 0)


_k4 = pl.kernel(
    _k4_body,
    out_type=_f32((N, D)),
    scratch_types=[
        pltpu.VMEM((CAP,), jnp.int32),
        pltpu.VMEM((NCHS, CH), jnp.int32),
        pltpu.VMEM((NCHS, CH), jnp.int32),
        pltpu.VMEM((2, CH, D), jnp.float32),
        pltpu.VMEM((RJ, D), jnp.float32),
        pltpu.VMEM((RJ, D), jnp.float32),
        pltpu.VMEM((RJ, D), jnp.float32),
        pltpu.VMEM((RJ, D), jnp.float32),
        pltpu.VMEM((RJ, D), jnp.float32),
        pltpu.VMEM((L,), jnp.int32),
        pltpu.VMEM((ST,), jnp.float32),
        pltpu.VMEM((ST,), jnp.float32),
        pltpu.SemaphoreType.DMA,
        pltpu.SemaphoreType.DMA,
        pltpu.VMEM_SHARED((HALF + 8, D), jnp.float32),
    ],
    **_MESH,
)


def _k5_body(lists_hbm, counts_hbm, cp_hbm, dinv_hbm, b_hbm, zer_hbm, out_hbm,
             listb, sidx, gidx, rows, u4c, cpc, ob, bb, cntv, dinv_v,
             sem_g, sem_s, acc):
    c = lax.axis_index("c")
    s = lax.axis_index("s")
    iot = lax.iota(jnp.int32, L)
    st = jnp.minimum(s * ST, HALF - ST)
    gbase = c * HALF + st

    pltpu.sync_copy(zer_hbm, acc.at[pl.ds(st, ST)])
    plsc.subcore_barrier()
    _spmm_phase(lists_hbm, counts_hbm, [cp_hbm], [acc],
                listb, sidx, gidx, [rows], cntv, sem_g, sem_s, c, s, iot)
    plsc.subcore_barrier()

    pltpu.sync_copy(dinv_hbm.at[pl.ds(gbase, ST)], dinv_v)
    pltpu.sync_copy(b_hbm, bb)
    zero16 = jnp.zeros((L,), jnp.float32)
    zidx = jnp.zeros((L,), jnp.int32)

    def epi(jj, carry):
        pltpu.sync_copy(acc.at[pl.ds(st + jj * RJ, RJ)], u4c)
        pltpu.sync_copy(cp_hbm.at[pl.ds(gbase + jj * RJ, RJ)], cpc)

        def row(r, carry2):
            ri = jj * RJ + r
            dv = plsc.load_gather(dinv_v, [zidx + ri])
            for k in range(D // L):
                sl = pl.ds(k * L, L)
                v = dv * (u4c[r, sl] + cpc[r, sl]) + bb[sl]
                ob[r, sl] = jnp.maximum(v, zero16)
            return carry2

        lax.fori_loop(0, RJ, row, 0)
        pltpu.sync_copy(ob, out_hbm.at[pl.ds(gbase + jj * RJ, RJ)])
        return carry

    lax.fori_loop(0, ST // RJ, epi, 0)


_k5 = pl.kernel(
    _k5_body,
    out_type=_f32((N, D)),
    scratch_types=[
        pltpu.VMEM((CAP,), jnp.int32),
        pltpu.VMEM((NCHS, CH), jnp.int32),
        pltpu.VMEM((NCHS, CH), jnp.int32),
        pltpu.VMEM((2, CH, D), jnp.float32),
        pltpu.VMEM((RJ, D), jnp.float32),
        pltpu.VMEM((RJ, D), jnp.float32),
        pltpu.VMEM((RJ, D), jnp.float32),
        pltpu.VMEM((D,), jnp.float32),
        pltpu.VMEM((L,), jnp.int32),
        pltpu.VMEM((ST,), jnp.float32),
        pltpu.SemaphoreType.DMA,
        pltpu.SemaphoreType.DMA,
        pltpu.VMEM_SHARED((HALF + 8, D), jnp.float32),
    ],
    **_MESH,
)


def kernel(x, edge_index, W, b, zeroth_w, fst_w, snd_w, trd_w):
    src = edge_index[0].astype(jnp.int32)
    dst = edge_index[1].astype(jnp.int32)
    wcat = jnp.concatenate([W, jnp.roll(W, -1, axis=0)], axis=1)
    zer = jnp.zeros((ST, D), jnp.float32)

    table = _k1a(src, dst)
    lists, counts, hists = _k1b(src, dst, table)
    zy, fy, syr = _k2a(x, wcat, zeroth_w, fst_w, snd_w, trd_w)
    deg, dinv = _k2b(hists)
    u1, u2 = _k3(lists, counts, fy, syr, zer)
    cp = _k4(lists, counts, u2, zy, u1, syr, deg, dinv, zer)
    out = _k5(lists, counts, cp, dinv, b, zer)
    return out


# speedup vs baseline: 6.2778x; 1.0914x over previous
import jax
import jax.numpy as jnp
from jax import lax
from jax.experimental import pallas as pl
from jax.experimental.pallas import tpu as pltpu
from jax.experimental.pallas import tpu_sc as plsc

N = 10000
E = 160000
E2 = 2 * E
D = 128
NC, NS, L = 2, 16, 16
NW = NC * NS
HALF = N // 2
SLICE = E2 // NW
CH = 128
CAP = 10240
NCHS = CAP // CH
TBL = N * N
DUMP = HALF
ST = 320
RJ = 32
RB = 3

_MESH = dict(
    mesh=plsc.VectorSubcoreMesh(
        core_axis_name="c", subcore_axis_name="s", num_cores=NC, num_subcores=NS),
    compiler_params=pltpu.CompilerParams(needs_layout_passes=False),
)


def _f32(shape):
    return jax.ShapeDtypeStruct(shape, jnp.float32)


def _i32(shape):
    return jax.ShapeDtypeStruct(shape, jnp.int32)


def _extract_i32(vec, idx, iot):
    return lax.reduce_max(jnp.where(iot == idx, vec, 0), axes=(0,))


def _k1a_body(src_hbm, dst_hbm, table_hbm, srcb, dstb, idxb, valb, sem):
    c = lax.axis_index("c")
    s = lax.axis_index("s")
    e0 = (c * NS + s) * SLICE
    swap = c == 1
    iot = lax.iota(jnp.int32, L)
    NB = 16

    pltpu.sync_copy(src_hbm.at[pl.ds(s * SLICE, SLICE)], srcb)
    pltpu.sync_copy(dst_hbm.at[pl.ds(s * SLICE, SLICE)], dstb)

    nchunk = (SLICE + CH - 1) // CH

    def fire(j, slot):
        start = jnp.minimum(j * CH, SLICE - CH)
        for k in range(CH // L):
            off = start + k * L
            sv = srcb[pl.ds(off, L)]
            dv = dstb[pl.ds(off, L)]
            usv = jnp.where(swap, dv, sv)
            udv = jnp.where(swap, sv, dv)
            idxb[slot, pl.ds(k * L, L)] = usv * N + udv
            valb[slot, pl.ds(k * L, L)] = e0 + off + iot
        pltpu.async_copy(valb.at[slot], table_hbm.at[idxb.at[slot]], sem)

    def body(j, carry):
        slot = lax.rem(j, NB)

        @pl.when(j >= NB)
        def _():
            pltpu.make_async_copy(valb.at[slot], table_hbm.at[idxb.at[slot]], sem).wait()

        fire(j, slot)
        return carry

    lax.fori_loop(0, nchunk, body, 0)
    for k in range(NB):
        @pl.when(k < nchunk)
        def _(k=k):
            pltpu.make_async_copy(valb.at[k], table_hbm.at[idxb.at[k]], sem).wait()


_k1a = pl.kernel(
    _k1a_body,
    out_type=_i32((TBL,)),
    scratch_types=[
        pltpu.VMEM((SLICE,), jnp.int32),
        pltpu.VMEM((SLICE,), jnp.int32),
        pltpu.VMEM((16, CH), jnp.int32),
        pltpu.VMEM((16, CH), jnp.int32),
        pltpu.SemaphoreType.DMA,
    ],
    **_MESH,
)


def _k1b_body(src_hbm, dst_hbm, table_hbm, lists_hbm, counts_hbm, hists_hbm,
              srcb, dstb, keyb, winb, listA, listB, hist, cntv, sem):
    c = lax.axis_index("c")
    s = lax.axis_index("s")
    t = c * NS + s
    e0 = t * SLICE
    swap = c == 1
    iot = lax.iota(jnp.int32, L)
    NB = 12
    zero16 = jnp.zeros((L,), jnp.float32)
    one16 = jnp.ones((L,), jnp.float32)

    pltpu.sync_copy(src_hbm.at[pl.ds(s * SLICE, SLICE)], srcb)
    pltpu.sync_copy(dst_hbm.at[pl.ds(s * SLICE, SLICE)], dstb)

    def zh(i, carry):
        hist[pl.ds(i * L, L)] = zero16
        return carry

    lax.fori_loop(0, N // L, zh, 0)

    nchunk = (SLICE + CH - 1) // CH

    def usud(off):
        sv = srcb[pl.ds(off, L)]
        dv = dstb[pl.ds(off, L)]
        usv = jnp.where(swap, dv, sv)
        udv = jnp.where(swap, sv, dv)
        return usv, udv

    def fire(j, slot):
        start = j * CH
        for k in range(CH // L):
            off = start + k * L
            usv, udv = usud(off)
            valid = (off + iot) < SLICE
            keyb[slot, pl.ds(k * L, L)] = jnp.where(valid, usv * N + udv, 0)
        pltpu.async_copy(table_hbm.at[keyb.at[slot]], winb.at[slot], sem)

    for j in range(NB):
        fire(j, j)

    def body(j, carry):
        cntA, cntB = carry
        slot = lax.rem(j, NB)
        pltpu.make_async_copy(table_hbm.at[keyb.at[slot]], winb.at[slot], sem).wait()
        start = j * CH
        for k in range(CH // L):
            off = start + k * L
            usv, udv = usud(off)
            valid = (off + iot) < SLICE
            ev = e0 + off + iot
            winv = winb[slot, pl.ds(k * L, L)]
            leader = valid & (winv == ev) & (usv != udv)
            usv_c = jnp.where(leader, usv, 0)
            plsc.addupdate_scatter(hist, [usv_c], one16, mask=leader)
            in_b = usv >= HALF
            packed = ((usv - jnp.where(in_b, HALF, 0)) << 16) | udv
            mA = leader & jnp.logical_not(in_b)
            mB = leader & in_b
            plsc.store_compressed(listA.at[pl.ds(cntA, L)], packed, mask=mA)
            plsc.store_compressed(listB.at[pl.ds(cntB, L)], packed, mask=mB)
            pcA = plsc.all_reduce_population_count(mA)
            pcB = plsc.all_reduce_population_count(mB)
            if pcA.ndim:
                pcA = lax.reduce_max(pcA, axes=(0,))
                pcB = lax.reduce_max(pcB, axes=(0,))
            cntA = cntA + pcA
            cntB = cntB + pcB

        @pl.when(j + NB < nchunk)
        def _():
            fire(j + NB, slot)

        return cntA, cntB

    cntA, cntB = lax.fori_loop(0, nchunk, body, (jnp.int32(0), jnp.int32(0)))

    cntv[...] = jnp.where(iot == 0, cntA, jnp.where(iot == 1, cntB, 0))
    pltpu.sync_copy(cntv, counts_hbm.at[t])
    pltpu.sync_copy(listA, lists_hbm.at[t, 0])
    pltpu.sync_copy(listB, lists_hbm.at[t, 1])
    pltpu.sync_copy(hist, hists_hbm.at[t])


_k1b = pl.kernel(
    _k1b_body,
    out_type=(_i32((NW, 2, CAP)), _i32((NW, L)), _f32((NW, N))),
    scratch_types=[
        pltpu.VMEM((SLICE,), jnp.int32),
        pltpu.VMEM((SLICE,), jnp.int32),
        pltpu.VMEM((12, CH), jnp.int32),
        pltpu.VMEM((12, CH), jnp.int32),
        pltpu.VMEM((CAP,), jnp.int32),
        pltpu.VMEM((CAP,), jnp.int32),
        pltpu.VMEM((N,), jnp.float32),
        pltpu.VMEM((L,), jnp.int32),
        pltpu.SemaphoreType.DMA,
    ],
    **_MESH,
)


def _k2a_body(zw, fw, sw, tw, x_ref, w_ref, zy_ref, fy_ref, syr_ref):
    z0 = jnp.maximum(zw[0, 0], 0.0)
    f0 = jnp.maximum(fw[0, 0], 0.0)
    s0 = jnp.maximum(sw[0, 0], 0.0)
    t0 = jnp.maximum(tw[0, 0], 0.0)
    tot = z0 + f0 + s0 + t0 + 1e-6
    z, f, sc = z0 / tot, f0 / tot, s0 / tot
    xw = jnp.dot(x_ref[...], w_ref[...], preferred_element_type=jnp.float32)
    y = xw[:, :D]
    yr = xw[:, D:]
    zy_ref[...] = z * y
    fy_ref[...] = f * y
    syr_ref[...] = sc * yr


def _k2a(x, wcat, zw, fw, sw, tw):
    bm = 1000
    return pl.pallas_call(
        _k2a_body,
        grid=(N // bm,),
        in_specs=[
            pl.BlockSpec(memory_space=pltpu.SMEM),
            pl.BlockSpec(memory_space=pltpu.SMEM),
            pl.BlockSpec(memory_space=pltpu.SMEM),
            pl.BlockSpec(memory_space=pltpu.SMEM),
            pl.BlockSpec((bm, D), lambda i: (i, 0)),
            pl.BlockSpec((D, 2 * D), lambda i: (0, 0)),
        ],
        out_specs=[
            pl.BlockSpec((bm, D), lambda i: (i, 0)),
            pl.BlockSpec((bm, D), lambda i: (i, 0)),
            pl.BlockSpec((bm, D), lambda i: (i, 0)),
        ],
        out_shape=[_f32((N, D)), _f32((N, D)), _f32((N, D))],
    )(zw, fw, sw, tw, x, wcat)


def _k2b_body(h_ref, deg_ref, dinv_ref):
    deg = jnp.sum(h_ref[...], axis=0)
    deg_ref[...] = deg
    dinv_ref[...] = lax.rsqrt(deg + 1.0)


def _k2b(hists):
    return pl.pallas_call(
        _k2b_body,
        out_shape=[_f32((N,)), _f32((N,))],
    )(hists)


def _spmm_segment(tabs, accs, rows_list, listb, sidx, gidx, cnt, nch,
                  sem_g, sem_s, iot):
    nt = len(tabs)

    def fire_g(j, slot):
        for k in range(CH // L):
            off = j * CH + k * L
            pk = listb[pl.ds(off, L)]
            valid = (off + iot) < cnt
            sidx[slot, pl.ds(k * L, L)] = jnp.where(valid, pk >> 16, DUMP)
            gidx[slot, pl.ds(k * L, L)] = jnp.where(valid, pk & 0xFFFF, 0)
        for i in range(nt):
            pltpu.async_copy(tabs[i].at[gidx.at[slot]], rows_list[i].at[slot], sem_g)

    def wait_g(slot):
        for i in range(nt):
            pltpu.make_async_copy(tabs[i].at[gidx.at[slot]], rows_list[i].at[slot], sem_g).wait()

    def fire_s(slot):
        for i in range(nt):
            pltpu.async_copy(rows_list[i].at[slot], accs[i].at[sidx.at[slot]], sem_s, add=True)

    def wait_s(slot):
        for i in range(nt):
            pltpu.make_async_copy(rows_list[i].at[slot], accs[i].at[sidx.at[slot]], sem_s).wait()

    for p in range(RB - 1):
        @pl.when(p < nch)
        def _(p=p):
            fire_g(p, p)

    def body(j, carry):
        slot = lax.rem(j, RB)
        wait_g(slot)
        fire_s(slot)

        @pl.when(j >= 1)
        def _():
            wait_s(lax.rem(j - 1, RB))

        @pl.when(j + (RB - 1) < nch)
        def _():
            fire_g(j + (RB - 1), lax.rem(j + (RB - 1), RB))

        return carry

    lax.fori_loop(0, nch, body, 0)

    @pl.when(nch > 0)
    def _():
        wait_s(lax.rem(nch - 1, RB))


def _spmm_phase(lists_hbm, counts_hbm, tabs, accs, listb, sidx, gidx, rows_list,
                cntv, sem_g, sem_s, c, s, iot):
    for k in range(2):
        tseg = 2 * s + k
        pltpu.sync_copy(lists_hbm.at[tseg, c], listb)
        pltpu.sync_copy(counts_hbm.at[tseg], cntv)
        cnt = _extract_i32(cntv[...], c, iot)
        nch = (cnt + CH - 1) // CH
        _spmm_segment(tabs, accs, rows_list, listb, sidx, gidx, cnt, nch,
                      sem_g, sem_s, iot)


def _k3_body(lists_hbm, counts_hbm, fy_hbm, syr_hbm, zer_hbm, u1_hbm, u2_hbm,
             listb, sidx, gidx, rows, cntv, sem_g, sem_s, acc):
    c = lax.axis_index("c")
    s = lax.axis_index("s")
    iot = lax.iota(jnp.int32, L)
    st = jnp.minimum(s * ST, HALF - ST)
    gbase = c * HALF + st

    for tab_hbm, out_hbm in ((fy_hbm, u1_hbm), (syr_hbm, u2_hbm)):
        pltpu.sync_copy(zer_hbm, acc.at[pl.ds(st, ST)])
        plsc.subcore_barrier()
        _spmm_phase(lists_hbm, counts_hbm, [tab_hbm], [acc],
                    listb, sidx, gidx, [rows], cntv, sem_g, sem_s, c, s, iot)
        plsc.subcore_barrier()
        pltpu.sync_copy(acc.at[pl.ds(st, ST)], out_hbm.at[pl.ds(gbase, ST)])
        plsc.subcore_barrier()


_k3 = pl.kernel(
    _k3_body,
    out_type=(_f32((N, D)), _f32((N, D))),
    scratch_types=[
        pltpu.VMEM((CAP,), jnp.int32),
        pltpu.VMEM((RB, CH), jnp.int32),
        pltpu.VMEM((RB, CH), jnp.int32),
        pltpu.VMEM((RB, CH, D), jnp.float32),
        pltpu.VMEM((L,), jnp.int32),
        pltpu.SemaphoreType.DMA,
        pltpu.SemaphoreType.DMA,
        pltpu.VMEM_SHARED((HALF + 8, D), jnp.float32),
    ],
    **_MESH,
)


def _k4_body(lists_hbm, counts_hbm, u2_hbm, zy_hbm, u1_hbm, syr_hbm, deg_hbm,
             dinv_hbm, zer_hbm, cp_hbm, listb, sidx, gidx, rows, u3c, zyc, u1c,
             syrc, cpb, cntv, deg_v, dinv_v, sem_g, sem_s, acc):
    c = lax.axis_index("c")
    s = lax.axis_index("s")
    iot = lax.iota(jnp.int32, L)
    st = jnp.minimum(s * ST, HALF - ST)
    gbase = c * HALF + st

    pltpu.sync_copy(zer_hbm, acc.at[pl.ds(st, ST)])
    plsc.subcore_barrier()
    _spmm_phase(lists_hbm, counts_hbm, [u2_hbm], [acc],
                listb, sidx, gidx, [rows], cntv, sem_g, sem_s, c, s, iot)
    plsc.subcore_barrier()

    pltpu.sync_copy(deg_hbm.at[pl.ds(gbase, ST)], deg_v)
    pltpu.sync_copy(dinv_hbm.at[pl.ds(gbase, ST)], dinv_v)
    zidx = jnp.zeros((L,), jnp.int32)

    def epi(jj, carry):
        pltpu.sync_copy(acc.at[pl.ds(st + jj * RJ, RJ)], u3c)
        pltpu.sync_copy(zy_hbm.at[pl.ds(gbase + jj * RJ, RJ)], zyc)
        pltpu.sync_copy(u1_hbm.at[pl.ds(gbase + jj * RJ, RJ)], u1c)
        pltpu.sync_copy(syr_hbm.at[pl.ds(gbase + jj * RJ, RJ)], syrc)

        def row(r, carry2):
            ri = jj * RJ + r
            dg = plsc.load_gather(deg_v, [zidx + ri])
            dv = plsc.load_gather(dinv_v, [zidx + ri])
            for k in range(D // L):
                sl = pl.ds(k * L, L)
                v = zyc[r, sl] + u1c[r, sl] + u3c[r, sl] - dg * syrc[r, sl]
                cpb[r, sl] = dv * v
            return carry2

        lax.fori_loop(0, RJ, row, 0)
        pltpu.sync_copy(cpb, cp_hbm.at[pl.ds(gbase + jj * RJ, RJ)])
        return carry

    lax.fori_loop(0, ST // RJ, epi, 0)


_k4 = pl.kernel(
    _k4_body,
    out_type=_f32((N, D)),
    scratch_types=[
        pltpu.VMEM((CAP,), jnp.int32),
        pltpu.VMEM((RB, CH), jnp.int32),
        pltpu.VMEM((RB, CH), jnp.int32),
        pltpu.VMEM((RB, CH, D), jnp.float32),
        pltpu.VMEM((RJ, D), jnp.float32),
        pltpu.VMEM((RJ, D), jnp.float32),
        pltpu.VMEM((RJ, D), jnp.float32),
        pltpu.VMEM((RJ, D), jnp.float32),
        pltpu.VMEM((RJ, D), jnp.float32),
        pltpu.VMEM((L,), jnp.int32),
        pltpu.VMEM((ST,), jnp.float32),
        pltpu.VMEM((ST,), jnp.float32),
        pltpu.SemaphoreType.DMA,
        pltpu.SemaphoreType.DMA,
        pltpu.VMEM_SHARED((HALF + 8, D), jnp.float32),
    ],
    **_MESH,
)


def _k5_body(lists_hbm, counts_hbm, cp_hbm, dinv_hbm, b_hbm, zer_hbm, out_hbm,
             listb, sidx, gidx, rows, u4c, cpc, ob, bb, cntv, dinv_v,
             sem_g, sem_s, acc):
    c = lax.axis_index("c")
    s = lax.axis_index("s")
    iot = lax.iota(jnp.int32, L)
    st = jnp.minimum(s * ST, HALF - ST)
    gbase = c * HALF + st

    pltpu.sync_copy(zer_hbm, acc.at[pl.ds(st, ST)])
    plsc.subcore_barrier()
    _spmm_phase(lists_hbm, counts_hbm, [cp_hbm], [acc],
                listb, sidx, gidx, [rows], cntv, sem_g, sem_s, c, s, iot)
    plsc.subcore_barrier()

    pltpu.sync_copy(dinv_hbm.at[pl.ds(gbase, ST)], dinv_v)
    pltpu.sync_copy(b_hbm, bb)
    zero16 = jnp.zeros((L,), jnp.float32)
    zidx = jnp.zeros((L,), jnp.int32)

    def epi(jj, carry):
        pltpu.sync_copy(acc.at[pl.ds(st + jj * RJ, RJ)], u4c)
        pltpu.sync_copy(cp_hbm.at[pl.ds(gbase + jj * RJ, RJ)], cpc)

        def row(r, carry2):
            ri = jj * RJ + r
            dv = plsc.load_gather(dinv_v, [zidx + ri])
            for k in range(D // L):
                sl = pl.ds(k * L, L)
                v = dv * (u4c[r, sl] + cpc[r, sl]) + bb[sl]
                ob[r, sl] = jnp.maximum(v, zero16)
            return carry2

        lax.fori_loop(0, RJ, row, 0)
        pltpu.sync_copy(ob, out_hbm.at[pl.ds(gbase + jj * RJ, RJ)])
        return carry

    lax.fori_loop(0, ST // RJ, epi, 0)


_k5 = pl.kernel(
    _k5_body,
    out_type=_f32((N, D)),
    scratch_types=[
        pltpu.VMEM((CAP,), jnp.int32),
        pltpu.VMEM((RB, CH), jnp.int32),
        pltpu.VMEM((RB, CH), jnp.int32),
        pltpu.VMEM((RB, CH, D), jnp.float32),
        pltpu.VMEM((RJ, D), jnp.float32),
        pltpu.VMEM((RJ, D), jnp.float32),
        pltpu.VMEM((RJ, D), jnp.float32),
        pltpu.VMEM((D,), jnp.float32),
        pltpu.VMEM((L,), jnp.int32),
        pltpu.VMEM((ST,), jnp.float32),
        pltpu.SemaphoreType.DMA,
        pltpu.SemaphoreType.DMA,
        pltpu.VMEM_SHARED((HALF + 8, D), jnp.float32),
    ],
    **_MESH,
)


def kernel(x, edge_index, W, b, zeroth_w, fst_w, snd_w, trd_w):
    src = edge_index[0].astype(jnp.int32)
    dst = edge_index[1].astype(jnp.int32)
    wcat = jnp.concatenate([W, jnp.roll(W, -1, axis=0)], axis=1)
    zer = jnp.zeros((ST, D), jnp.float32)

    table = _k1a(src, dst)
    lists, counts, hists = _k1b(src, dst, table)
    zy, fy, syr = _k2a(x, wcat, zeroth_w, fst_w, snd_w, trd_w)
    deg, dinv = _k2b(hists)
    u1, u2 = _k3(lists, counts, fy, syr, zer)
    cp = _k4(lists, counts, u2, zy, u1, syr, deg, dinv, zer)
    out = _k5(lists, counts, cp, dinv, b, zer)
    return out


# speedup vs baseline: 6.3245x; 1.0074x over previous
import jax
import jax.numpy as jnp
from jax import lax
from jax.experimental import pallas as pl
from jax.experimental.pallas import tpu as pltpu
from jax.experimental.pallas import tpu_sc as plsc

N = 10000
E = 160000
E2 = 2 * E
D = 128
NC, NS, L = 2, 16, 16
NW = NC * NS
HALF = N // 2
SLICE = E2 // NW
CH = 128
CAP = 10240
NCHS = CAP // CH
TBL = N * N
DUMP = HALF
ST = 320
RJ = 32
RB = 4

_MESH = dict(
    mesh=plsc.VectorSubcoreMesh(
        core_axis_name="c", subcore_axis_name="s", num_cores=NC, num_subcores=NS),
    compiler_params=pltpu.CompilerParams(needs_layout_passes=False),
)


def _f32(shape):
    return jax.ShapeDtypeStruct(shape, jnp.float32)


def _i32(shape):
    return jax.ShapeDtypeStruct(shape, jnp.int32)


def _extract_i32(vec, idx, iot):
    return lax.reduce_max(jnp.where(iot == idx, vec, 0), axes=(0,))


def _k1a_body(src_hbm, dst_hbm, table_hbm, srcb, dstb, idxb, valb, sem):
    c = lax.axis_index("c")
    s = lax.axis_index("s")
    e0 = (c * NS + s) * SLICE
    swap = c == 1
    iot = lax.iota(jnp.int32, L)
    NB = 16

    pltpu.sync_copy(src_hbm.at[pl.ds(s * SLICE, SLICE)], srcb)
    pltpu.sync_copy(dst_hbm.at[pl.ds(s * SLICE, SLICE)], dstb)

    nchunk = (SLICE + CH - 1) // CH

    def fire(j, slot):
        start = jnp.minimum(j * CH, SLICE - CH)
        for k in range(CH // L):
            off = start + k * L
            sv = srcb[pl.ds(off, L)]
            dv = dstb[pl.ds(off, L)]
            usv = jnp.where(swap, dv, sv)
            udv = jnp.where(swap, sv, dv)
            idxb[slot, pl.ds(k * L, L)] = usv * N + udv
            valb[slot, pl.ds(k * L, L)] = e0 + off + iot
        pltpu.async_copy(valb.at[slot], table_hbm.at[idxb.at[slot]], sem)

    def body(j, carry):
        slot = lax.rem(j, NB)

        @pl.when(j >= NB)
        def _():
            pltpu.make_async_copy(valb.at[slot], table_hbm.at[idxb.at[slot]], sem).wait()

        fire(j, slot)
        return carry

    lax.fori_loop(0, nchunk, body, 0)
    for k in range(NB):
        @pl.when(k < nchunk)
        def _(k=k):
            pltpu.make_async_copy(valb.at[k], table_hbm.at[idxb.at[k]], sem).wait()


_k1a = pl.kernel(
    _k1a_body,
    out_type=_i32((TBL,)),
    scratch_types=[
        pltpu.VMEM((SLICE,), jnp.int32),
        pltpu.VMEM((SLICE,), jnp.int32),
        pltpu.VMEM((16, CH), jnp.int32),
        pltpu.VMEM((16, CH), jnp.int32),
        pltpu.SemaphoreType.DMA,
    ],
    **_MESH,
)


def _k1b_body(src_hbm, dst_hbm, table_hbm, lists_hbm, counts_hbm, hists_hbm,
              srcb, dstb, keyb, winb, listA, listB, hist, cntv, sem):
    c = lax.axis_index("c")
    s = lax.axis_index("s")
    t = c * NS + s
    e0 = t * SLICE
    swap = c == 1
    iot = lax.iota(jnp.int32, L)
    NB = 12
    zero16 = jnp.zeros((L,), jnp.float32)
    one16 = jnp.ones((L,), jnp.float32)

    pltpu.sync_copy(src_hbm.at[pl.ds(s * SLICE, SLICE)], srcb)
    pltpu.sync_copy(dst_hbm.at[pl.ds(s * SLICE, SLICE)], dstb)

    def zh(i, carry):
        hist[pl.ds(i * L, L)] = zero16
        return carry

    lax.fori_loop(0, N // L, zh, 0)

    nchunk = (SLICE + CH - 1) // CH

    def usud(off):
        sv = srcb[pl.ds(off, L)]
        dv = dstb[pl.ds(off, L)]
        usv = jnp.where(swap, dv, sv)
        udv = jnp.where(swap, sv, dv)
        return usv, udv

    def fire(j, slot):
        start = j * CH
        for k in range(CH // L):
            off = start + k * L
            usv, udv = usud(off)
            valid = (off + iot) < SLICE
            keyb[slot, pl.ds(k * L, L)] = jnp.where(valid, usv * N + udv, 0)
        pltpu.async_copy(table_hbm.at[keyb.at[slot]], winb.at[slot], sem)

    for j in range(NB):
        fire(j, j)

    def body(j, carry):
        cntA, cntB = carry
        slot = lax.rem(j, NB)
        pltpu.make_async_copy(table_hbm.at[keyb.at[slot]], winb.at[slot], sem).wait()
        start = j * CH
        for k in range(CH // L):
            off = start + k * L
            usv, udv = usud(off)
            valid = (off + iot) < SLICE
            ev = e0 + off + iot
            winv = winb[slot, pl.ds(k * L, L)]
            leader = valid & (winv == ev) & (usv != udv)
            usv_c = jnp.where(leader, usv, 0)
            plsc.addupdate_scatter(hist, [usv_c], one16, mask=leader)
            in_b = usv >= HALF
            packed = ((usv - jnp.where(in_b, HALF, 0)) << 16) | udv
            mA = leader & jnp.logical_not(in_b)
            mB = leader & in_b
            plsc.store_compressed(listA.at[pl.ds(cntA, L)], packed, mask=mA)
            plsc.store_compressed(listB.at[pl.ds(cntB, L)], packed, mask=mB)
            pcA = plsc.all_reduce_population_count(mA)
            pcB = plsc.all_reduce_population_count(mB)
            if pcA.ndim:
                pcA = lax.reduce_max(pcA, axes=(0,))
                pcB = lax.reduce_max(pcB, axes=(0,))
            cntA = cntA + pcA
            cntB = cntB + pcB

        @pl.when(j + NB < nchunk)
        def _():
            fire(j + NB, slot)

        return cntA, cntB

    cntA, cntB = lax.fori_loop(0, nchunk, body, (jnp.int32(0), jnp.int32(0)))

    cntv[...] = jnp.where(iot == 0, cntA, jnp.where(iot == 1, cntB, 0))
    pltpu.sync_copy(cntv, counts_hbm.at[t])
    pltpu.sync_copy(listA, lists_hbm.at[t, 0])
    pltpu.sync_copy(listB, lists_hbm.at[t, 1])
    pltpu.sync_copy(hist, hists_hbm.at[t])


_k1b = pl.kernel(
    _k1b_body,
    out_type=(_i32((NW, 2, CAP)), _i32((NW, L)), _f32((NW, N))),
    scratch_types=[
        pltpu.VMEM((SLICE,), jnp.int32),
        pltpu.VMEM((SLICE,), jnp.int32),
        pltpu.VMEM((12, CH), jnp.int32),
        pltpu.VMEM((12, CH), jnp.int32),
        pltpu.VMEM((CAP,), jnp.int32),
        pltpu.VMEM((CAP,), jnp.int32),
        pltpu.VMEM((N,), jnp.float32),
        pltpu.VMEM((L,), jnp.int32),
        pltpu.SemaphoreType.DMA,
    ],
    **_MESH,
)


def _k2a_body(zw, fw, sw, tw, x_ref, w_ref, zy_ref, fy_ref, syr_ref):
    z0 = jnp.maximum(zw[0, 0], 0.0)
    f0 = jnp.maximum(fw[0, 0], 0.0)
    s0 = jnp.maximum(sw[0, 0], 0.0)
    t0 = jnp.maximum(tw[0, 0], 0.0)
    tot = z0 + f0 + s0 + t0 + 1e-6
    z, f, sc = z0 / tot, f0 / tot, s0 / tot
    xw = jnp.dot(x_ref[...], w_ref[...], preferred_element_type=jnp.float32)
    y = xw[:, :D]
    yr = xw[:, D:]
    zy_ref[...] = z * y
    fy_ref[...] = f * y
    syr_ref[...] = sc * yr


def _k2a(x, wcat, zw, fw, sw, tw):
    bm = 1000
    return pl.pallas_call(
        _k2a_body,
        grid=(N // bm,),
        in_specs=[
            pl.BlockSpec(memory_space=pltpu.SMEM),
            pl.BlockSpec(memory_space=pltpu.SMEM),
            pl.BlockSpec(memory_space=pltpu.SMEM),
            pl.BlockSpec(memory_space=pltpu.SMEM),
            pl.BlockSpec((bm, D), lambda i: (i, 0)),
            pl.BlockSpec((D, 2 * D), lambda i: (0, 0)),
        ],
        out_specs=[
            pl.BlockSpec((bm, D), lambda i: (i, 0)),
            pl.BlockSpec((bm, D), lambda i: (i, 0)),
            pl.BlockSpec((bm, D), lambda i: (i, 0)),
        ],
        out_shape=[_f32((N, D)), _f32((N, D)), _f32((N, D))],
    )(zw, fw, sw, tw, x, wcat)


def _k2b_body(h_ref, deg_ref, dinv_ref):
    deg = jnp.sum(h_ref[...], axis=0)
    deg_ref[...] = deg
    dinv_ref[...] = lax.rsqrt(deg + 1.0)


def _k2b(hists):
    return pl.pallas_call(
        _k2b_body,
        out_shape=[_f32((N,)), _f32((N,))],
    )(hists)


def _spmm_segment(tabs, accs, rows_list, listb, sidx, gidx, cnt, nch,
                  sem_g, sem_s, iot):
    nt = len(tabs)

    def fire_g(j, slot):
        for k in range(CH // L):
            off = j * CH + k * L
            pk = listb[pl.ds(off, L)]
            valid = (off + iot) < cnt
            sidx[slot, pl.ds(k * L, L)] = jnp.where(valid, pk >> 16, DUMP)
            gidx[slot, pl.ds(k * L, L)] = jnp.where(valid, pk & 0xFFFF, 0)
        for i in range(nt):
            pltpu.async_copy(tabs[i].at[gidx.at[slot]], rows_list[i].at[slot], sem_g)

    def wait_g(slot):
        for i in range(nt):
            pltpu.make_async_copy(tabs[i].at[gidx.at[slot]], rows_list[i].at[slot], sem_g).wait()

    def fire_s(slot):
        for i in range(nt):
            pltpu.async_copy(rows_list[i].at[slot], accs[i].at[sidx.at[slot]], sem_s, add=True)

    def wait_s(slot):
        for i in range(nt):
            pltpu.make_async_copy(rows_list[i].at[slot], accs[i].at[sidx.at[slot]], sem_s).wait()

    for p in range(RB - 1):
        @pl.when(p < nch)
        def _(p=p):
            fire_g(p, p)

    def body(j, carry):
        slot = lax.rem(j, RB)
        wait_g(slot)
        fire_s(slot)

        @pl.when(j >= 1)
        def _():
            wait_s(lax.rem(j - 1, RB))

        @pl.when(j + (RB - 1) < nch)
        def _():
            fire_g(j + (RB - 1), lax.rem(j + (RB - 1), RB))

        return carry

    lax.fori_loop(0, nch, body, 0)

    @pl.when(nch > 0)
    def _():
        wait_s(lax.rem(nch - 1, RB))


def _spmm_phase(lists_hbm, counts_hbm, tabs, accs, listb, sidx, gidx, rows_list,
                cntv, sem_g, sem_s, c, s, iot):
    for k in range(2):
        tseg = 2 * s + k
        pltpu.sync_copy(lists_hbm.at[tseg, c], listb)
        pltpu.sync_copy(counts_hbm.at[tseg], cntv)
        cnt = _extract_i32(cntv[...], c, iot)
        nch = (cnt + CH - 1) // CH
        _spmm_segment(tabs, accs, rows_list, listb, sidx, gidx, cnt, nch,
                      sem_g, sem_s, iot)


def _k3_body(lists_hbm, counts_hbm, fy_hbm, syr_hbm, zer_hbm, u1_hbm, u2_hbm,
             listb, sidx, gidx, rows, cntv, sem_g, sem_s, acc):
    c = lax.axis_index("c")
    s = lax.axis_index("s")
    iot = lax.iota(jnp.int32, L)
    st = jnp.minimum(s * ST, HALF - ST)
    gbase = c * HALF + st

    for tab_hbm, out_hbm in ((fy_hbm, u1_hbm), (syr_hbm, u2_hbm)):
        pltpu.sync_copy(zer_hbm, acc.at[pl.ds(st, ST)])
        plsc.subcore_barrier()
        _spmm_phase(lists_hbm, counts_hbm, [tab_hbm], [acc],
                    listb, sidx, gidx, [rows], cntv, sem_g, sem_s, c, s, iot)
        plsc.subcore_barrier()
        pltpu.sync_copy(acc.at[pl.ds(st, ST)], out_hbm.at[pl.ds(gbase, ST)])
        plsc.subcore_barrier()


_k3 = pl.kernel(
    _k3_body,
    out_type=(_f32((N, D)), _f32((N, D))),
    scratch_types=[
        pltpu.VMEM((CAP,), jnp.int32),
        pltpu.VMEM((RB, CH), jnp.int32),
        pltpu.VMEM((RB, CH), jnp.int32),
        pltpu.VMEM((RB, CH, D), jnp.float32),
        pltpu.VMEM((L,), jnp.int32),
        pltpu.SemaphoreType.DMA,
        pltpu.SemaphoreType.DMA,
        pltpu.VMEM_SHARED((HALF + 8, D), jnp.float32),
    ],
    **_MESH,
)


def _k4_body(lists_hbm, counts_hbm, u2_hbm, zy_hbm, u1_hbm, syr_hbm, deg_hbm,
             dinv_hbm, zer_hbm, cp_hbm, listb, sidx, gidx, rows, cntv,
             deg_v, dinv_v, sem_g, sem_s, acc):
    c = lax.axis_index("c")
    s = lax.axis_index("s")
    iot = lax.iota(jnp.int32, L)
    st = jnp.minimum(s * ST, HALF - ST)
    gbase = c * HALF + st

    pltpu.sync_copy(zer_hbm, acc.at[pl.ds(st, ST)])
    plsc.subcore_barrier()
    _spmm_phase(lists_hbm, counts_hbm, [u2_hbm], [acc],
                listb, sidx, gidx, [rows], cntv, sem_g, sem_s, c, s, iot)
    plsc.subcore_barrier()

    pltpu.sync_copy(deg_hbm.at[pl.ds(gbase, ST)], deg_v)
    pltpu.sync_copy(dinv_hbm.at[pl.ds(gbase, ST)], dinv_v)
    zidx = jnp.zeros((L,), jnp.int32)
    u3c = rows.at[0, pl.ds(0, RJ)]
    zyc = rows.at[0, pl.ds(RJ, RJ)]
    u1c = rows.at[0, pl.ds(2 * RJ, RJ)]
    syrc = rows.at[0, pl.ds(3 * RJ, RJ)]
    cpb = rows.at[1, pl.ds(0, RJ)]

    def epi(jj, carry):
        pltpu.sync_copy(acc.at[pl.ds(st + jj * RJ, RJ)], u3c)
        pltpu.sync_copy(zy_hbm.at[pl.ds(gbase + jj * RJ, RJ)], zyc)
        pltpu.sync_copy(u1_hbm.at[pl.ds(gbase + jj * RJ, RJ)], u1c)
        pltpu.sync_copy(syr_hbm.at[pl.ds(gbase + jj * RJ, RJ)], syrc)

        def row(r, carry2):
            ri = jj * RJ + r
            dg = plsc.load_gather(deg_v, [zidx + ri])
            dv = plsc.load_gather(dinv_v, [zidx + ri])
            for k in range(D // L):
                sl = pl.ds(k * L, L)
                v = zyc[r, sl] + u1c[r, sl] + u3c[r, sl] - dg * syrc[r, sl]
                cpb[r, sl] = dv * v
            return carry2

        lax.fori_loop(0, RJ, row, 0)
        pltpu.sync_copy(cpb, cp_hbm.at[pl.ds(gbase + jj * RJ, RJ)])
        return carry

    lax.fori_loop(0, ST // RJ, epi, 0)


_k4 = pl.kernel(
    _k4_body,
    out_type=_f32((N, D)),
    scratch_types=[
        pltpu.VMEM((CAP,), jnp.int32),
        pltpu.VMEM((RB, CH), jnp.int32),
        pltpu.VMEM((RB, CH), jnp.int32),
        pltpu.VMEM((RB, CH, D), jnp.float32),
        pltpu.VMEM((L,), jnp.int32),
        pltpu.VMEM((ST,), jnp.float32),
        pltpu.VMEM((ST,), jnp.float32),
        pltpu.SemaphoreType.DMA,
        pltpu.SemaphoreType.DMA,
        pltpu.VMEM_SHARED((HALF + 8, D), jnp.float32),
    ],
    **_MESH,
)


def _k5_body(lists_hbm, counts_hbm, cp_hbm, dinv_hbm, b_hbm, zer_hbm, out_hbm,
             listb, sidx, gidx, rows, bb, cntv, dinv_v,
             sem_g, sem_s, acc):
    c = lax.axis_index("c")
    s = lax.axis_index("s")
    iot = lax.iota(jnp.int32, L)
    st = jnp.minimum(s * ST, HALF - ST)
    gbase = c * HALF + st

    pltpu.sync_copy(zer_hbm, acc.at[pl.ds(st, ST)])
    plsc.subcore_barrier()
    _spmm_phase(lists_hbm, counts_hbm, [cp_hbm], [acc],
                listb, sidx, gidx, [rows], cntv, sem_g, sem_s, c, s, iot)
    plsc.subcore_barrier()

    pltpu.sync_copy(dinv_hbm.at[pl.ds(gbase, ST)], dinv_v)
    pltpu.sync_copy(b_hbm, bb)
    zero16 = jnp.zeros((L,), jnp.float32)
    zidx = jnp.zeros((L,), jnp.int32)
    u4c = rows.at[0, pl.ds(0, RJ)]
    cpc = rows.at[0, pl.ds(RJ, RJ)]
    ob = rows.at[0, pl.ds(2 * RJ, RJ)]

    def epi(jj, carry):
        pltpu.sync_copy(acc.at[pl.ds(st + jj * RJ, RJ)], u4c)
        pltpu.sync_copy(cp_hbm.at[pl.ds(gbase + jj * RJ, RJ)], cpc)

        def row(r, carry2):
            ri = jj * RJ + r
            dv = plsc.load_gather(dinv_v, [zidx + ri])
            for k in range(D // L):
                sl = pl.ds(k * L, L)
                v = dv * (u4c[r, sl] + cpc[r, sl]) + bb[sl]
                ob[r, sl] = jnp.maximum(v, zero16)
            return carry2

        lax.fori_loop(0, RJ, row, 0)
        pltpu.sync_copy(ob, out_hbm.at[pl.ds(gbase + jj * RJ, RJ)])
        return carry

    lax.fori_loop(0, ST // RJ, epi, 0)


_k5 = pl.kernel(
    _k5_body,
    out_type=_f32((N, D)),
    scratch_types=[
        pltpu.VMEM((CAP,), jnp.int32),
        pltpu.VMEM((RB, CH), jnp.int32),
        pltpu.VMEM((RB, CH), jnp.int32),
        pltpu.VMEM((RB, CH, D), jnp.float32),
        pltpu.VMEM((D,), jnp.float32),
        pltpu.VMEM((L,), jnp.int32),
        pltpu.VMEM((ST,), jnp.float32),
        pltpu.SemaphoreType.DMA,
        pltpu.SemaphoreType.DMA,
        pltpu.VMEM_SHARED((HALF + 8, D), jnp.float32),
    ],
    **_MESH,
)


def kernel(x, edge_index, W, b, zeroth_w, fst_w, snd_w, trd_w):
    src = edge_index[0].astype(jnp.int32)
    dst = edge_index[1].astype(jnp.int32)
    wcat = jnp.concatenate([W, jnp.roll(W, -1, axis=0)], axis=1)
    zer = jnp.zeros((ST, D), jnp.float32)

    table = _k1a(src, dst)
    lists, counts, hists = _k1b(src, dst, table)
    zy, fy, syr = _k2a(x, wcat, zeroth_w, fst_w, snd_w, trd_w)
    deg, dinv = _k2b(hists)
    u1, u2 = _k3(lists, counts, fy, syr, zer)
    cp = _k4(lists, counts, u2, zy, u1, syr, deg, dinv, zer)
    out = _k5(lists, counts, cp, dinv, b, zer)
    return out
